# baseline (speedup 1.0000x reference)
import functools

import jax
import jax.numpy as jnp
from jax import lax
from jax.experimental import pallas as pl
from jax.experimental.pallas import tpu as pltpu
from jax.experimental.pallas import tpu_sc as plsc

_NBUF = 4


@functools.lru_cache(maxsize=None)
def _build(B, H, V, D, num_cores, num_subcores):
    n_workers = num_cores * num_subcores
    NT = B // 128
    t_per_w = NT // n_workers
    n_chunks = t_per_w * H
    n_groups = n_chunks // _NBUF
    b_per_w = t_per_w * 128 * H
    NA = D // 8
    assert n_groups * _NBUF == n_chunks

    mesh = plsc.VectorSubcoreMesh(core_axis_name="c", subcore_axis_name="s")

    @functools.partial(
        pl.kernel,
        mesh=mesh,
        compiler_params=pltpu.CompilerParams(use_tc_tiling_on_sc=False,
                                             needs_layout_passes=False),
        out_type=jax.ShapeDtypeStruct((H, NA, NT, 8, 128), jnp.float32),
        scratch_types=[
            pltpu.VMEM((b_per_w,), jnp.int32),
            pltpu.VMEM((_NBUF, 128), jnp.int32),
            pltpu.VMEM((_NBUF, 128, D), jnp.float32),
            pltpu.VMEM((_NBUF, NA, 8, 128), jnp.float32),
            pltpu.SemaphoreType.DMA,
            pltpu.SemaphoreType.DMA,
        ],
    )
    def k(table_hbm, idx_hbm, out_hbm, blk_v, idx_v, gbuf_v, tbuf_v,
          gsem, wsem):
        wid = lax.axis_index("s") * num_cores + lax.axis_index("c")
        base = wid * b_per_w
        t0 = wid * t_per_w
        pltpu.sync_copy(idx_hbm.at[pl.ds(base, b_per_w)], blk_v)

        lane = lax.iota(jnp.int32, 16)

        def build_idx(c, b):
            h = c % H
            tl = c // H
            off = tl * (128 * H) + h
            for kk in range(8):
                addr = (lane + 16 * kk) * H + off
                idx_v[b, pl.ds(16 * kk, 16)] = plsc.load_gather(blk_v, [addr])

        def fire_gather(b):
            pltpu.async_copy(table_hbm.at[idx_v.at[b]], gbuf_v.at[b], gsem)

        def wait_gather(b):
            pltpu.make_async_copy(table_hbm.at[idx_v.at[b]],
                                  gbuf_v.at[b], gsem).wait()

        def transpose(b):
            for d in range(D):
                col = lane * 0 + d
                for kk in range(8):
                    row = lane + 16 * kk
                    tbuf_v[b, d // 8, d % 8, pl.ds(16 * kk, 16)] = (
                        plsc.load_gather(gbuf_v.at[b], [row, col]))

        def fire_writeback(c, b):
            h = c % H
            t = t0 + c // H
            for a in range(NA):
                pltpu.async_copy(tbuf_v.at[b, a], out_hbm.at[h, a, t], wsem)

        def wait_writeback(b):
            for a in range(NA):
                pltpu.make_async_copy(tbuf_v.at[b, a],
                                      out_hbm.at[0, a, 0], wsem).wait()

        for b in range(_NBUF):
            build_idx(b, b)
            fire_gather(b)

        def group(g, carry):
            for b in range(_NBUF):
                c = g * _NBUF + b
                wait_gather(b)
                transpose(b)
                fire_writeback(c, b)
            for b in range(_NBUF):
                wait_writeback(b)
                build_idx((g + 1) * _NBUF + b, b)
                fire_gather(b)
            return carry

        lax.fori_loop(0, n_groups - 1, group, 0)

        g = n_groups - 1
        for b in range(_NBUF):
            wait_gather(b)
            transpose(b)
            fire_writeback(g * _NBUF + b, b)
        for b in range(_NBUF):
            wait_writeback(b)

    return k


def kernel(x, table):
    bsz, hist = x.shape
    vocab, dim = table.shape
    idx = x.reshape(bsz * hist).astype(jnp.int32)
    info = plsc.get_sparse_core_info()
    o5 = _build(bsz, hist, vocab, dim, info.num_cores, info.num_subcores)(
        table, idx)
    return o5.transpose(2, 4, 0, 1, 3).reshape(bsz, hist, dim)


# speedup vs baseline: 1.2562x; 1.2562x over previous
import functools

import jax
import jax.numpy as jnp
from jax import lax
from jax.experimental import pallas as pl
from jax.experimental.pallas import tpu as pltpu
from jax.experimental.pallas import tpu_sc as plsc

_NBUF = 4


@functools.lru_cache(maxsize=None)
def _build(B, H, V, D, num_cores, num_subcores):
    n_workers = num_cores * num_subcores
    NT = B // 128
    t_per_w = NT // n_workers
    n_chunks = t_per_w * H
    n_groups = n_chunks // _NBUF
    b_per_w = t_per_w * 128 * H
    NA = D // 8
    assert n_groups * _NBUF == n_chunks

    mesh = plsc.VectorSubcoreMesh(core_axis_name="c", subcore_axis_name="s")

    @functools.partial(
        pl.kernel,
        mesh=mesh,
        compiler_params=pltpu.CompilerParams(use_tc_tiling_on_sc=False,
                                             needs_layout_passes=False),
        out_type=jax.ShapeDtypeStruct((H, NA, NT, 8, 128), jnp.float32),
        scratch_types=[
            pltpu.VMEM((b_per_w,), jnp.int32),
            pltpu.VMEM((_NBUF, 128), jnp.int32),
            pltpu.VMEM((_NBUF, 128, D), jnp.float32),
            pltpu.VMEM((_NBUF, D, 129), jnp.float32),
            pltpu.SemaphoreType.DMA,
            pltpu.SemaphoreType.DMA,
        ],
    )
    def k(table_hbm, idx_hbm, out_hbm, blk_v, idx_v, gbuf_v, tbuf_v,
          gsem, wsem):
        wid = lax.axis_index("s") * num_cores + lax.axis_index("c")
        base = wid * b_per_w
        t0 = wid * t_per_w
        pltpu.sync_copy(idx_hbm.at[pl.ds(base, b_per_w)], blk_v)

        lane = lax.iota(jnp.int32, 16)

        def build_idx(c, b):
            h = c % H
            tl = c // H
            off = tl * (128 * H) + h
            for kk in range(8):
                addr = (lane + 16 * kk) * H + off
                idx_v[b, pl.ds(16 * kk, 16)] = plsc.load_gather(blk_v, [addr])

        def fire_gather(b):
            pltpu.async_copy(table_hbm.at[idx_v.at[b]], gbuf_v.at[b], gsem)

        def wait_gather(b):
            pltpu.make_async_copy(table_hbm.at[idx_v.at[b]],
                                  gbuf_v.at[b], gsem).wait()

        lane_hi = lane + 16

        def transpose(b):
            for j in range(128):
                jv = lane * 0 + j
                lo = gbuf_v[b, j, pl.ds(0, 16)]
                hi = gbuf_v[b, j, pl.ds(16, 16)]
                plsc.store_scatter(tbuf_v.at[b], [lane, jv], lo)
                plsc.store_scatter(tbuf_v.at[b], [lane_hi, jv], hi)

        def fire_writeback(c, b):
            h = c % H
            t = t0 + c // H
            for a in range(NA):
                pltpu.async_copy(
                    tbuf_v.at[b, pl.ds(8 * a, 8), pl.ds(0, 128)],
                    out_hbm.at[h, a, t], wsem)

        def wait_writeback(b):
            for a in range(NA):
                pltpu.make_async_copy(
                    tbuf_v.at[b, pl.ds(8 * a, 8), pl.ds(0, 128)],
                    out_hbm.at[0, a, 0], wsem).wait()

        for b in range(_NBUF):
            build_idx(b, b)
            fire_gather(b)

        def group(g, carry):
            for b in range(_NBUF):
                c = g * _NBUF + b
                wait_gather(b)
                transpose(b)
                fire_writeback(c, b)
            for b in range(_NBUF):
                wait_writeback(b)
                build_idx((g + 1) * _NBUF + b, b)
                fire_gather(b)
            return carry

        lax.fori_loop(0, n_groups - 1, group, 0)

        g = n_groups - 1
        for b in range(_NBUF):
            wait_gather(b)
            transpose(b)
            fire_writeback(g * _NBUF + b, b)
        for b in range(_NBUF):
            wait_writeback(b)

    return k


def kernel(x, table):
    bsz, hist = x.shape
    vocab, dim = table.shape
    idx = x.reshape(bsz * hist).astype(jnp.int32)
    info = plsc.get_sparse_core_info()
    o5 = _build(bsz, hist, vocab, dim, info.num_cores, info.num_subcores)(
        table, idx)
    return o5.transpose(2, 4, 0, 1, 3).reshape(bsz, hist, dim)
